# baseline (speedup 1.0000x reference)
import functools

import jax
import jax.numpy as jnp
from jax import lax
from jax.experimental import pallas as pl
from jax.experimental.pallas import tpu as pltpu
from jax.experimental.pallas import tpu_sc as plsc

B = 64
N = 155
U = 60
D = 256
ENC = 256
NUM_EDGE_TYPES = 4
TIMESTEPS = 2
E_PER_TYPE = 80000
NODES = B * (N + 1)
TOTAL_E = NUM_EDGE_TYPES * E_PER_TYPE

HALF = D // 2
ROW_BLK = 128
NUM_ROW_BLKS = NODES // ROW_BLK
H_ROWS = 2 * NUM_EDGE_TYPES * NODES

CHUNK = 128
NS = 16
GRP = 32
CHUNKS_PER_TILE = 160
NGRP = CHUNKS_PER_TILE // GRP
E_PAD = CHUNKS_PER_TILE * CHUNK * NS
M_ROWS = NODES + 128
ZROWS_PER_TILE = M_ROWS // NS
OROWS_PER_TILE = NODES // NS


def _prep_body(ls_ref, um_ref, ete_ref, enc_ref, wp_ref, bp_ref, g_ref,
               lp_ref, x0_ref):
    ls = ls_ref[0]
    um = um_ref[0, 0]
    vm = ls * um[None, :]
    mx = jnp.maximum(jnp.max(vm, axis=1, keepdims=True), 0.0)
    en = jnp.exp(vm - mx)
    denom = jnp.sum(en, axis=1, keepdims=True) + jnp.exp(-mx)
    s = en / denom * um[None, :]
    lp = s / (jnp.sum(s, axis=1, keepdims=True) + 1e-13)
    lp_ref[0] = lp
    r0 = jnp.max(lp, axis=1, keepdims=True)
    q = jnp.dot(lp, enc_ref[0], preferred_element_type=jnp.float32)
    init = (jnp.dot(ete_ref[0], wp_ref[0:D], preferred_element_type=jnp.float32)
            + r0 * wp_ref[D][None, :]
            + jnp.dot(q, wp_ref[D + 1:], preferred_element_type=jnp.float32)
            + bp_ref[0][None, :])
    x0_ref[0] = jnp.concatenate([init, g_ref[:]], axis=0)


def _prep(linking_scores, um3, entity_type_embeddings, encoder_outputs,
          W_proj, b_proj2, global_emb):
    return pl.pallas_call(
        _prep_body,
        grid=(B,),
        in_specs=[
            pl.BlockSpec((1, N, U), lambda b: (b, 0, 0)),
            pl.BlockSpec((1, 1, U), lambda b: (b, 0, 0)),
            pl.BlockSpec((1, N, D), lambda b: (b, 0, 0)),
            pl.BlockSpec((1, U, ENC), lambda b: (b, 0, 0)),
            pl.BlockSpec((D + ENC + 1, D), lambda b: (0, 0)),
            pl.BlockSpec((1, D), lambda b: (0, 0)),
            pl.BlockSpec((1, D), lambda b: (0, 0)),
        ],
        out_specs=[
            pl.BlockSpec((1, N, U), lambda b: (b, 0, 0)),
            pl.BlockSpec((1, N + 1, D), lambda b: (b, 0, 0)),
        ],
        out_shape=[
            jax.ShapeDtypeStruct((B, N, U), jnp.float32),
            jax.ShapeDtypeStruct((B, N + 1, D), jnp.float32),
        ],
    )(linking_scores, um3, entity_type_embeddings, encoder_outputs,
      W_proj, b_proj2, global_emb)


def _hproj_body(x_ref, w_ref, h_ref):
    h_ref[...] = jnp.dot(x_ref[...], w_ref[0, 0],
                         preferred_element_type=jnp.float32)


def _hproj(x, W4):
    return pl.pallas_call(
        _hproj_body,
        grid=(2, NUM_EDGE_TYPES, NUM_ROW_BLKS),
        in_specs=[
            pl.BlockSpec((ROW_BLK, D), lambda c, e, r: (r, 0)),
            pl.BlockSpec((1, 1, D, HALF), lambda c, e, r: (c, e, 0, 0)),
        ],
        out_specs=pl.BlockSpec(
            (ROW_BLK, HALF),
            lambda c, e, r: (c * NUM_EDGE_TYPES * NUM_ROW_BLKS
                             + e * NUM_ROW_BLKS + r, 0)),
        out_shape=jax.ShapeDtypeStruct((H_ROWS, HALF), jnp.float32),
    )(x, W4)


def _sc_agg_body(h3, src_idx, dst_idx, zeros_hbm, out,
                 sidx_v, didx_v, rows0_v, rows1_v, m_sh, sem0, sem1,
                 ssem0, ssem1):
    c = lax.axis_index("c")
    s = lax.axis_index("s")
    pltpu.sync_copy(zeros_hbm.at[pl.ds(s * ZROWS_PER_TILE, ZROWS_PER_TILE)],
                    m_sh.at[pl.ds(s * ZROWS_PER_TILE, ZROWS_PER_TILE)])
    plsc.subcore_barrier()

    def outer(g, carry):
        pltpu.sync_copy(src_idx.at[c].at[s].at[pl.ds(g * GRP, GRP)], sidx_v)
        pltpu.sync_copy(dst_idx.at[s].at[pl.ds(g * GRP, GRP)], didx_v)

        pltpu.async_copy(h3.at[sidx_v.at[0]], rows0_v, sem0)
        pltpu.async_copy(h3.at[sidx_v.at[1]], rows1_v, sem1)

        def body(j2, c2):
            j = 2 * j2
            pltpu.make_async_copy(h3.at[sidx_v.at[j]], rows0_v, sem0).wait()
            pltpu.async_copy(rows0_v, m_sh.at[didx_v.at[j]], ssem0, add=True)
            pltpu.make_async_copy(h3.at[sidx_v.at[j + 1]], rows1_v, sem1).wait()
            pltpu.async_copy(rows1_v, m_sh.at[didx_v.at[j + 1]], ssem1, add=True)
            jn0 = jnp.minimum(j + 2, GRP - 1)
            jn1 = jnp.minimum(j + 3, GRP - 1)
            pltpu.make_async_copy(rows0_v, m_sh.at[didx_v.at[j]], ssem0).wait()
            pltpu.async_copy(h3.at[sidx_v.at[jn0]], rows0_v, sem0)
            pltpu.make_async_copy(rows1_v, m_sh.at[didx_v.at[j + 1]], ssem1).wait()
            pltpu.async_copy(h3.at[sidx_v.at[jn1]], rows1_v, sem1)
            return c2

        r = lax.fori_loop(0, GRP // 2, body, carry)
        pltpu.make_async_copy(h3.at[sidx_v.at[0]], rows0_v, sem0).wait()
        pltpu.make_async_copy(h3.at[sidx_v.at[1]], rows1_v, sem1).wait()
        return r

    lax.fori_loop(0, NGRP, outer, 0)
    plsc.subcore_barrier()
    pltpu.sync_copy(m_sh.at[pl.ds(s * OROWS_PER_TILE, OROWS_PER_TILE)],
                    out.at[c].at[pl.ds(s * OROWS_PER_TILE, OROWS_PER_TILE)])


@functools.cache
def _make_sc_agg():
    @functools.partial(
        pl.kernel,
        mesh=plsc.VectorSubcoreMesh(core_axis_name="c", subcore_axis_name="s"),
        out_type=jax.ShapeDtypeStruct((2, NODES, HALF), jnp.float32),
        scratch_types=[
            pltpu.VMEM((GRP, CHUNK), jnp.int32),
            pltpu.VMEM((GRP, CHUNK), jnp.int32),
            pltpu.VMEM((CHUNK, HALF), jnp.float32),
            pltpu.VMEM((CHUNK, HALF), jnp.float32),
            pltpu.VMEM_SHARED((M_ROWS, HALF), jnp.float32),
            pltpu.SemaphoreType.DMA,
            pltpu.SemaphoreType.DMA,
            pltpu.SemaphoreType.DMA,
            pltpu.SemaphoreType.DMA,
        ],
    )
    def _sc_agg(h3, src_idx, dst_idx, zeros_hbm, out,
                sidx_v, didx_v, rows0_v, rows1_v, m_sh, sem0, sem1,
                ssem0, ssem1):
        _sc_agg_body(h3, src_idx, dst_idx, zeros_hbm, out,
                     sidx_v, didx_v, rows0_v, rows1_v, m_sh, sem0, sem1,
                     ssem0, ssem1)

    return _sc_agg


def _gru_body(mlo_ref, mhi_ref, x_ref, wih_ref, whh_ref, bih_ref, bhh_ref,
              wrel_ref, brel_ref, xn_ref, lg_ref, sg_ref):
    gi = (jnp.dot(mlo_ref[...], wih_ref[0:HALF], preferred_element_type=jnp.float32)
          + jnp.dot(mhi_ref[...], wih_ref[HALF:], preferred_element_type=jnp.float32)
          + bih_ref[0][None, :])
    gh = (jnp.dot(x_ref[...], whh_ref[...], preferred_element_type=jnp.float32)
          + bhh_ref[0][None, :])
    r = jax.nn.sigmoid(gi[:, 0:D] + gh[:, 0:D])
    z = jax.nn.sigmoid(gi[:, D:2 * D] + gh[:, D:2 * D])
    n = jnp.tanh(gi[:, 2 * D:] + r * gh[:, 2 * D:])
    xn = (1.0 - z) * n + z * x_ref[...]
    xn_ref[...] = xn
    lg = jnp.sum(xn * wrel_ref[0][None, :], axis=1, keepdims=True) + brel_ref[0, 0]
    lg_ref[...] = jnp.broadcast_to(lg, (ROW_BLK, ROW_BLK))
    sg_ref[...] = jax.nn.sigmoid(lg_ref[...])


def _gru(m2, x, W_ihT, W_hhT, b_ih2, b_hh2, w_relT, b_rel2):
    return pl.pallas_call(
        _gru_body,
        grid=(NUM_ROW_BLKS,),
        in_specs=[
            pl.BlockSpec((ROW_BLK, HALF), lambda r: (r, 0)),
            pl.BlockSpec((ROW_BLK, HALF), lambda r: (r, 0)),
            pl.BlockSpec((ROW_BLK, D), lambda r: (r, 0)),
            pl.BlockSpec((D, 3 * D), lambda r: (0, 0)),
            pl.BlockSpec((D, 3 * D), lambda r: (0, 0)),
            pl.BlockSpec((1, 3 * D), lambda r: (0, 0)),
            pl.BlockSpec((1, 3 * D), lambda r: (0, 0)),
            pl.BlockSpec((1, D), lambda r: (0, 0)),
            pl.BlockSpec((1, 1), lambda r: (0, 0)),
        ],
        out_specs=[
            pl.BlockSpec((ROW_BLK, D), lambda r: (r, 0)),
            pl.BlockSpec((ROW_BLK, ROW_BLK), lambda r: (r, 0)),
            pl.BlockSpec((ROW_BLK, ROW_BLK), lambda r: (r, 0)),
        ],
        out_shape=[
            jax.ShapeDtypeStruct((NODES, D), jnp.float32),
            jax.ShapeDtypeStruct((NODES, ROW_BLK), jnp.float32),
            jax.ShapeDtypeStruct((NODES, ROW_BLK), jnp.float32),
        ],
    )(m2[0], m2[1], x, W_ihT, W_hhT, b_ih2, b_hh2, w_relT, b_rel2)


def kernel(encoder_outputs, entity_type_embeddings, linking_scores,
           utterance_mask, edge_index_0, edge_index_1, edge_index_2,
           edge_index_3, W_proj, b_proj, global_emb, W_edge, W_ih, W_hh,
           b_ih, b_hh, W_rel, b_rel):
    edge_indices = [edge_index_0, edge_index_1, edge_index_2, edge_index_3]
    src_all = jnp.concatenate(
        [edge_indices[e][0] + e * NODES for e in range(NUM_EDGE_TYPES)])
    dst_all = jnp.concatenate([edge_indices[e][1] for e in range(NUM_EDGE_TYPES)])
    npad = E_PAD - TOTAL_E
    pad_i = jnp.arange(npad, dtype=jnp.int32)
    src_p = jnp.concatenate([src_all, pad_i % CHUNK])
    dst_p = jnp.concatenate([dst_all, NODES + (pad_i % 128)])
    src2 = jnp.stack([src_p, src_p + NUM_EDGE_TYPES * NODES]).reshape(
        2, NS, CHUNKS_PER_TILE, CHUNK)
    dst2 = dst_p.reshape(NS, CHUNKS_PER_TILE, CHUNK)
    zeros_hbm = jnp.zeros((M_ROWS, HALF), jnp.float32)

    um3 = utterance_mask.reshape(B, 1, U)
    b_proj2 = b_proj.reshape(1, D)
    g2 = global_emb.reshape(1, D)
    W4 = W_edge.reshape(NUM_EDGE_TYPES, D, 2, HALF).transpose(2, 0, 1, 3)
    W_ihT = W_ih.T
    W_hhT = W_hh.T
    b_ih2 = b_ih.reshape(1, 3 * D)
    b_hh2 = b_hh.reshape(1, 3 * D)
    w_relT = W_rel.reshape(1, D)
    b_rel2 = b_rel.reshape(1, 1)

    lp, x0 = _prep(linking_scores, um3, entity_type_embeddings,
                   encoder_outputs, W_proj, b_proj2, g2)
    x = x0.reshape(NODES, D)

    for _ in range(TIMESTEPS):
        h3 = _hproj(x, W4)
        m2 = _make_sc_agg()(h3, src2, dst2, zeros_hbm)
        x, lg, sg = _gru(m2, x, W_ihT, W_hhT, b_ih2, b_hh2, w_relT, b_rel2)

    logits = lg[:, 0].reshape(B, N + 1, 1)[:, :N]
    sig = sg[:, 0].reshape(B, N + 1, 1)[:, :N]
    return (sig, logits, lp)


# speedup vs baseline: 1.1160x; 1.1160x over previous
import functools

import jax
import jax.numpy as jnp
from jax import lax
from jax.experimental import pallas as pl
from jax.experimental.pallas import tpu as pltpu
from jax.experimental.pallas import tpu_sc as plsc

B = 64
N = 155
U = 60
D = 256
ENC = 256
NUM_EDGE_TYPES = 4
TIMESTEPS = 2
E_PER_TYPE = 80000
NODES = B * (N + 1)
TOTAL_E = NUM_EDGE_TYPES * E_PER_TYPE

HALF = D // 2
ROW_BLK = 128
NUM_ROW_BLKS = NODES // ROW_BLK
H_ROWS = 2 * NUM_EDGE_TYPES * NODES

CHUNK = 128
NS = 16
GRP = 32
CHUNKS_PER_TILE = 160
NGRP = CHUNKS_PER_TILE // GRP
E_PAD = CHUNKS_PER_TILE * CHUNK * NS
M_ROWS = NODES + 128
ZROWS_PER_TILE = M_ROWS // NS
OROWS_PER_TILE = NODES // NS


def _prep_body(ls_ref, um_ref, ete_ref, enc_ref, wp_ref, bp_ref, g_ref,
               lp_ref, x0_ref):
    ls = ls_ref[0]
    um = um_ref[0, 0]
    vm = ls * um[None, :]
    mx = jnp.maximum(jnp.max(vm, axis=1, keepdims=True), 0.0)
    en = jnp.exp(vm - mx)
    denom = jnp.sum(en, axis=1, keepdims=True) + jnp.exp(-mx)
    s = en / denom * um[None, :]
    lp = s / (jnp.sum(s, axis=1, keepdims=True) + 1e-13)
    lp_ref[0] = lp
    r0 = jnp.max(lp, axis=1, keepdims=True)
    q = jnp.dot(lp, enc_ref[0], preferred_element_type=jnp.float32)
    init = (jnp.dot(ete_ref[0], wp_ref[0:D], preferred_element_type=jnp.float32)
            + r0 * wp_ref[D][None, :]
            + jnp.dot(q, wp_ref[D + 1:], preferred_element_type=jnp.float32)
            + bp_ref[0][None, :])
    x0_ref[0] = jnp.concatenate([init, g_ref[:]], axis=0)


def _prep(linking_scores, um3, entity_type_embeddings, encoder_outputs,
          W_proj, b_proj2, global_emb):
    return pl.pallas_call(
        _prep_body,
        grid=(B,),
        in_specs=[
            pl.BlockSpec((1, N, U), lambda b: (b, 0, 0)),
            pl.BlockSpec((1, 1, U), lambda b: (b, 0, 0)),
            pl.BlockSpec((1, N, D), lambda b: (b, 0, 0)),
            pl.BlockSpec((1, U, ENC), lambda b: (b, 0, 0)),
            pl.BlockSpec((D + ENC + 1, D), lambda b: (0, 0)),
            pl.BlockSpec((1, D), lambda b: (0, 0)),
            pl.BlockSpec((1, D), lambda b: (0, 0)),
        ],
        out_specs=[
            pl.BlockSpec((1, N, U), lambda b: (b, 0, 0)),
            pl.BlockSpec((1, N + 1, D), lambda b: (b, 0, 0)),
        ],
        out_shape=[
            jax.ShapeDtypeStruct((B, N, U), jnp.float32),
            jax.ShapeDtypeStruct((B, N + 1, D), jnp.float32),
        ],
    )(linking_scores, um3, entity_type_embeddings, encoder_outputs,
      W_proj, b_proj2, global_emb)


def _hproj_body(x_ref, w_ref, h_ref):
    h_ref[...] = jnp.dot(x_ref[...], w_ref[0, 0],
                         preferred_element_type=jnp.float32)


def _hproj(x, W4):
    return pl.pallas_call(
        _hproj_body,
        grid=(2, NUM_EDGE_TYPES, NUM_ROW_BLKS),
        in_specs=[
            pl.BlockSpec((ROW_BLK, D), lambda c, e, r: (r, 0)),
            pl.BlockSpec((1, 1, D, HALF), lambda c, e, r: (c, e, 0, 0)),
        ],
        out_specs=pl.BlockSpec(
            (ROW_BLK, HALF),
            lambda c, e, r: (c * NUM_EDGE_TYPES * NUM_ROW_BLKS
                             + e * NUM_ROW_BLKS + r, 0)),
        out_shape=jax.ShapeDtypeStruct((H_ROWS, HALF), jnp.float32),
    )(x, W4)


def _sc_agg_body(h3, src_idx, dst_idx, zeros_hbm, out,
                 sidx_v, didx_v, rows0_v, rows1_v, m_sh, sem0, sem1,
                 ssem0, ssem1):
    c = lax.axis_index("c")
    s = lax.axis_index("s")
    pltpu.sync_copy(zeros_hbm.at[pl.ds(s * ZROWS_PER_TILE, ZROWS_PER_TILE)],
                    m_sh.at[pl.ds(s * ZROWS_PER_TILE, ZROWS_PER_TILE)])
    plsc.subcore_barrier()

    def outer(g, carry):
        pltpu.sync_copy(src_idx.at[c].at[s].at[pl.ds(g * GRP, GRP)], sidx_v)
        pltpu.sync_copy(dst_idx.at[s].at[pl.ds(g * GRP, GRP)], didx_v)

        pltpu.async_copy(h3.at[sidx_v.at[0]], rows0_v, sem0)

        def body(j2, c2):
            j = 2 * j2
            pltpu.async_copy(h3.at[sidx_v.at[j + 1]], rows1_v, sem1)
            pltpu.make_async_copy(h3.at[sidx_v.at[j]], rows0_v, sem0).wait()
            pltpu.sync_copy(rows0_v, m_sh.at[didx_v.at[j]], add=True)
            jn = jnp.minimum(j + 2, GRP - 1)
            pltpu.async_copy(h3.at[sidx_v.at[jn]], rows0_v, sem0)
            pltpu.make_async_copy(h3.at[sidx_v.at[j + 1]], rows1_v, sem1).wait()
            pltpu.sync_copy(rows1_v, m_sh.at[didx_v.at[j + 1]], add=True)
            return c2

        r = lax.fori_loop(0, GRP // 2, body, carry)
        pltpu.make_async_copy(h3.at[sidx_v.at[0]], rows0_v, sem0).wait()
        return r

    lax.fori_loop(0, NGRP, outer, 0)
    plsc.subcore_barrier()
    pltpu.sync_copy(m_sh.at[pl.ds(s * OROWS_PER_TILE, OROWS_PER_TILE)],
                    out.at[c].at[pl.ds(s * OROWS_PER_TILE, OROWS_PER_TILE)])


@functools.cache
def _make_sc_agg():
    @functools.partial(
        pl.kernel,
        mesh=plsc.VectorSubcoreMesh(core_axis_name="c", subcore_axis_name="s"),
        out_type=jax.ShapeDtypeStruct((2, NODES, HALF), jnp.float32),
        scratch_types=[
            pltpu.VMEM((GRP, CHUNK), jnp.int32),
            pltpu.VMEM((GRP, CHUNK), jnp.int32),
            pltpu.VMEM((CHUNK, HALF), jnp.float32),
            pltpu.VMEM((CHUNK, HALF), jnp.float32),
            pltpu.VMEM_SHARED((M_ROWS, HALF), jnp.float32),
            pltpu.SemaphoreType.DMA,
            pltpu.SemaphoreType.DMA,
            pltpu.SemaphoreType.DMA,
            pltpu.SemaphoreType.DMA,
        ],
    )
    def _sc_agg(h3, src_idx, dst_idx, zeros_hbm, out,
                sidx_v, didx_v, rows0_v, rows1_v, m_sh, sem0, sem1,
                ssem0, ssem1):
        _sc_agg_body(h3, src_idx, dst_idx, zeros_hbm, out,
                     sidx_v, didx_v, rows0_v, rows1_v, m_sh, sem0, sem1,
                     ssem0, ssem1)

    return _sc_agg


def _gru_body(mlo_ref, mhi_ref, x_ref, wih_ref, whh_ref, bih_ref, bhh_ref,
              wrel_ref, brel_ref, xn_ref, lg_ref, sg_ref):
    gi = (jnp.dot(mlo_ref[...], wih_ref[0:HALF], preferred_element_type=jnp.float32)
          + jnp.dot(mhi_ref[...], wih_ref[HALF:], preferred_element_type=jnp.float32)
          + bih_ref[0][None, :])
    gh = (jnp.dot(x_ref[...], whh_ref[...], preferred_element_type=jnp.float32)
          + bhh_ref[0][None, :])
    r = jax.nn.sigmoid(gi[:, 0:D] + gh[:, 0:D])
    z = jax.nn.sigmoid(gi[:, D:2 * D] + gh[:, D:2 * D])
    n = jnp.tanh(gi[:, 2 * D:] + r * gh[:, 2 * D:])
    xn = (1.0 - z) * n + z * x_ref[...]
    xn_ref[...] = xn
    lg = jnp.sum(xn * wrel_ref[0][None, :], axis=1, keepdims=True) + brel_ref[0, 0]
    lg_ref[...] = jnp.broadcast_to(lg, (ROW_BLK, ROW_BLK))
    sg_ref[...] = jax.nn.sigmoid(lg_ref[...])


def _gru(m2, x, W_ihT, W_hhT, b_ih2, b_hh2, w_relT, b_rel2):
    return pl.pallas_call(
        _gru_body,
        grid=(NUM_ROW_BLKS,),
        in_specs=[
            pl.BlockSpec((ROW_BLK, HALF), lambda r: (r, 0)),
            pl.BlockSpec((ROW_BLK, HALF), lambda r: (r, 0)),
            pl.BlockSpec((ROW_BLK, D), lambda r: (r, 0)),
            pl.BlockSpec((D, 3 * D), lambda r: (0, 0)),
            pl.BlockSpec((D, 3 * D), lambda r: (0, 0)),
            pl.BlockSpec((1, 3 * D), lambda r: (0, 0)),
            pl.BlockSpec((1, 3 * D), lambda r: (0, 0)),
            pl.BlockSpec((1, D), lambda r: (0, 0)),
            pl.BlockSpec((1, 1), lambda r: (0, 0)),
        ],
        out_specs=[
            pl.BlockSpec((ROW_BLK, D), lambda r: (r, 0)),
            pl.BlockSpec((ROW_BLK, ROW_BLK), lambda r: (r, 0)),
            pl.BlockSpec((ROW_BLK, ROW_BLK), lambda r: (r, 0)),
        ],
        out_shape=[
            jax.ShapeDtypeStruct((NODES, D), jnp.float32),
            jax.ShapeDtypeStruct((NODES, ROW_BLK), jnp.float32),
            jax.ShapeDtypeStruct((NODES, ROW_BLK), jnp.float32),
        ],
    )(m2[0], m2[1], x, W_ihT, W_hhT, b_ih2, b_hh2, w_relT, b_rel2)


def kernel(encoder_outputs, entity_type_embeddings, linking_scores,
           utterance_mask, edge_index_0, edge_index_1, edge_index_2,
           edge_index_3, W_proj, b_proj, global_emb, W_edge, W_ih, W_hh,
           b_ih, b_hh, W_rel, b_rel):
    edge_indices = [edge_index_0, edge_index_1, edge_index_2, edge_index_3]
    src_all = jnp.concatenate(
        [edge_indices[e][0] + e * NODES for e in range(NUM_EDGE_TYPES)])
    dst_all = jnp.concatenate([edge_indices[e][1] for e in range(NUM_EDGE_TYPES)])
    npad = E_PAD - TOTAL_E
    pad_i = jnp.arange(npad, dtype=jnp.int32)
    src_p = jnp.concatenate([src_all, pad_i % CHUNK])
    dst_p = jnp.concatenate([dst_all, NODES + (pad_i % 128)])
    src2 = jnp.stack([src_p, src_p + NUM_EDGE_TYPES * NODES]).reshape(
        2, NS, CHUNKS_PER_TILE, CHUNK)
    dst2 = dst_p.reshape(NS, CHUNKS_PER_TILE, CHUNK)
    zeros_hbm = jnp.zeros((M_ROWS, HALF), jnp.float32)

    um3 = utterance_mask.reshape(B, 1, U)
    b_proj2 = b_proj.reshape(1, D)
    g2 = global_emb.reshape(1, D)
    W4 = W_edge.reshape(NUM_EDGE_TYPES, D, 2, HALF).transpose(2, 0, 1, 3)
    W_ihT = W_ih.T
    W_hhT = W_hh.T
    b_ih2 = b_ih.reshape(1, 3 * D)
    b_hh2 = b_hh.reshape(1, 3 * D)
    w_relT = W_rel.reshape(1, D)
    b_rel2 = b_rel.reshape(1, 1)

    lp, x0 = _prep(linking_scores, um3, entity_type_embeddings,
                   encoder_outputs, W_proj, b_proj2, g2)
    x = x0.reshape(NODES, D)

    for _ in range(TIMESTEPS):
        h3 = _hproj(x, W4)
        m2 = _make_sc_agg()(h3, src2, dst2, zeros_hbm)
        x, lg, sg = _gru(m2, x, W_ihT, W_hhT, b_ih2, b_hh2, w_relT, b_rel2)

    logits = lg[:, 0].reshape(B, N + 1, 1)[:, :N]
    sig = sg[:, 0].reshape(B, N + 1, 1)[:, :N]
    return (sig, logits, lp)


# speedup vs baseline: 2.3074x; 2.0675x over previous
import functools

import jax
import jax.numpy as jnp
from jax import lax
from jax.experimental import pallas as pl
from jax.experimental.pallas import tpu as pltpu
from jax.experimental.pallas import tpu_sc as plsc

B = 64
N = 155
U = 60
D = 256
ENC = 256
NUM_EDGE_TYPES = 4
TIMESTEPS = 2
E_PER_TYPE = 80000
NODES = B * (N + 1)
TOTAL_E = NUM_EDGE_TYPES * E_PER_TYPE

HALF = D // 2
ROW_BLK = 128
NUM_ROW_BLKS = NODES // ROW_BLK
H_ROWS = 2 * NUM_EDGE_TYPES * NODES

CHUNK = 128
NS = 16
GRP = 32
CHUNKS_PER_TILE = 160
NGRP = CHUNKS_PER_TILE // GRP
E_PAD = CHUNKS_PER_TILE * CHUNK * NS
M_ROWS = NODES + 128
ZROWS_PER_TILE = M_ROWS // NS
OROWS_PER_TILE = NODES // NS
G_RB = 1248


def _prep_body(ls_ref, um_ref, ete_ref, enc_ref, wp_ref, bp_ref, g_ref,
               lp_ref, x0_ref):
    ls = ls_ref[0]
    um = um_ref[0, 0]
    vm = ls * um[None, :]
    mx = jnp.maximum(jnp.max(vm, axis=1, keepdims=True), 0.0)
    en = jnp.exp(vm - mx)
    denom = jnp.sum(en, axis=1, keepdims=True) + jnp.exp(-mx)
    s = en / denom * um[None, :]
    lp = s / (jnp.sum(s, axis=1, keepdims=True) + 1e-13)
    lp_ref[0] = lp
    r0 = jnp.max(lp, axis=1, keepdims=True)
    q = jnp.dot(lp, enc_ref[0], preferred_element_type=jnp.float32)
    init = (jnp.dot(ete_ref[0], wp_ref[0:D], preferred_element_type=jnp.float32)
            + r0 * wp_ref[D][None, :]
            + jnp.dot(q, wp_ref[D + 1:], preferred_element_type=jnp.float32)
            + bp_ref[0][None, :])
    x0_ref[0] = jnp.concatenate([init, g_ref[:]], axis=0)


def _prep(linking_scores, um3, entity_type_embeddings, encoder_outputs,
          W_proj, b_proj2, global_emb):
    return pl.pallas_call(
        _prep_body,
        grid=(B,),
        in_specs=[
            pl.BlockSpec((1, N, U), lambda b: (b, 0, 0)),
            pl.BlockSpec((1, 1, U), lambda b: (b, 0, 0)),
            pl.BlockSpec((1, N, D), lambda b: (b, 0, 0)),
            pl.BlockSpec((1, U, ENC), lambda b: (b, 0, 0)),
            pl.BlockSpec((D + ENC + 1, D), lambda b: (0, 0)),
            pl.BlockSpec((1, D), lambda b: (0, 0)),
            pl.BlockSpec((1, D), lambda b: (0, 0)),
        ],
        out_specs=[
            pl.BlockSpec((1, N, U), lambda b: (b, 0, 0)),
            pl.BlockSpec((1, N + 1, D), lambda b: (b, 0, 0)),
        ],
        out_shape=[
            jax.ShapeDtypeStruct((B, N, U), jnp.float32),
            jax.ShapeDtypeStruct((B, N + 1, D), jnp.float32),
        ],
    )(linking_scores, um3, entity_type_embeddings, encoder_outputs,
      W_proj, b_proj2, global_emb)


HP_RB = 2496
HP_NR = NODES // HP_RB


def _hproj_body(x_ref, w_ref, h_ref):
    h_ref[...] = jnp.dot(x_ref[...], w_ref[0],
                         preferred_element_type=jnp.float32)


def _hproj(x, W8):
    return pl.pallas_call(
        _hproj_body,
        grid=(HP_NR, 2 * NUM_EDGE_TYPES),
        in_specs=[
            pl.BlockSpec((HP_RB, D), lambda r, k: (r, 0)),
            pl.BlockSpec((1, D, HALF), lambda r, k: (k, 0, 0)),
        ],
        out_specs=pl.BlockSpec(
            (HP_RB, HALF), lambda r, k: (k * HP_NR + r, 0)),
        out_shape=jax.ShapeDtypeStruct((H_ROWS, HALF), jnp.float32),
    )(x, W8)


def _sc_agg_body(h3, src_idx, dst_idx, zeros_hbm, out,
                 sidx_v, didx_v, rows0_v, rows1_v, m_sh, sem0, sem1,
                 ssem0, ssem1):
    c = lax.axis_index("c")
    s = lax.axis_index("s")
    pltpu.sync_copy(zeros_hbm.at[pl.ds(s * ZROWS_PER_TILE, ZROWS_PER_TILE)],
                    m_sh.at[pl.ds(s * ZROWS_PER_TILE, ZROWS_PER_TILE)])
    plsc.subcore_barrier()

    def outer(g, carry):
        pltpu.sync_copy(src_idx.at[c].at[s].at[pl.ds(g * GRP, GRP)], sidx_v)
        pltpu.sync_copy(dst_idx.at[s].at[pl.ds(g * GRP, GRP)], didx_v)

        pltpu.async_copy(h3.at[sidx_v.at[0]], rows0_v, sem0)

        def body(j2, c2):
            j = 2 * j2
            pltpu.async_copy(h3.at[sidx_v.at[j + 1]], rows1_v, sem1)
            pltpu.make_async_copy(h3.at[sidx_v.at[j]], rows0_v, sem0).wait()
            pltpu.sync_copy(rows0_v, m_sh.at[didx_v.at[j]], add=True)
            jn = jnp.minimum(j + 2, GRP - 1)
            pltpu.async_copy(h3.at[sidx_v.at[jn]], rows0_v, sem0)
            pltpu.make_async_copy(h3.at[sidx_v.at[j + 1]], rows1_v, sem1).wait()
            pltpu.sync_copy(rows1_v, m_sh.at[didx_v.at[j + 1]], add=True)
            return c2

        r = lax.fori_loop(0, GRP // 2, body, carry)
        pltpu.make_async_copy(h3.at[sidx_v.at[0]], rows0_v, sem0).wait()
        return r

    lax.fori_loop(0, NGRP, outer, 0)
    plsc.subcore_barrier()
    pltpu.sync_copy(m_sh.at[pl.ds(s * OROWS_PER_TILE, OROWS_PER_TILE)],
                    out.at[c].at[pl.ds(s * OROWS_PER_TILE, OROWS_PER_TILE)])


@functools.cache
def _make_sc_agg():
    @functools.partial(
        pl.kernel,
        mesh=plsc.VectorSubcoreMesh(core_axis_name="c", subcore_axis_name="s"),
        out_type=jax.ShapeDtypeStruct((2, NODES, HALF), jnp.float32),
        scratch_types=[
            pltpu.VMEM((GRP, CHUNK), jnp.int32),
            pltpu.VMEM((GRP, CHUNK), jnp.int32),
            pltpu.VMEM((CHUNK, HALF), jnp.float32),
            pltpu.VMEM((CHUNK, HALF), jnp.float32),
            pltpu.VMEM_SHARED((M_ROWS, HALF), jnp.float32),
            pltpu.SemaphoreType.DMA,
            pltpu.SemaphoreType.DMA,
            pltpu.SemaphoreType.DMA,
            pltpu.SemaphoreType.DMA,
        ],
    )
    def _sc_agg(h3, src_idx, dst_idx, zeros_hbm, out,
                sidx_v, didx_v, rows0_v, rows1_v, m_sh, sem0, sem1,
                ssem0, ssem1):
        _sc_agg_body(h3, src_idx, dst_idx, zeros_hbm, out,
                     sidx_v, didx_v, rows0_v, rows1_v, m_sh, sem0, sem1,
                     ssem0, ssem1)

    return _sc_agg


def _gru_body(mlo_ref, mhi_ref, x_ref, wih_ref, whh_ref, bih_ref, bhh_ref,
              wrel_ref, brel_ref, xn_ref, lg_ref, sg_ref):
    gi = (jnp.dot(mlo_ref[...], wih_ref[0:HALF], preferred_element_type=jnp.float32)
          + jnp.dot(mhi_ref[...], wih_ref[HALF:], preferred_element_type=jnp.float32)
          + bih_ref[0][None, :])
    gh = (jnp.dot(x_ref[...], whh_ref[...], preferred_element_type=jnp.float32)
          + bhh_ref[0][None, :])
    r = jax.nn.sigmoid(gi[:, 0:D] + gh[:, 0:D])
    z = jax.nn.sigmoid(gi[:, D:2 * D] + gh[:, D:2 * D])
    n = jnp.tanh(gi[:, 2 * D:] + r * gh[:, 2 * D:])
    xn = (1.0 - z) * n + z * x_ref[...]
    xn_ref[...] = xn
    lg = jnp.sum(xn * wrel_ref[0][None, :], axis=1, keepdims=True) + brel_ref[0, 0]
    lg_ref[...] = jnp.broadcast_to(lg, (G_RB, ROW_BLK))
    sg_ref[...] = jax.nn.sigmoid(lg_ref[...])


def _gru(m2, x, W_ihT, W_hhT, b_ih2, b_hh2, w_relT, b_rel2):
    return pl.pallas_call(
        _gru_body,
        grid=(NODES // G_RB,),
        in_specs=[
            pl.BlockSpec((G_RB, HALF), lambda r: (r, 0)),
            pl.BlockSpec((G_RB, HALF), lambda r: (r, 0)),
            pl.BlockSpec((G_RB, D), lambda r: (r, 0)),
            pl.BlockSpec((D, 3 * D), lambda r: (0, 0)),
            pl.BlockSpec((D, 3 * D), lambda r: (0, 0)),
            pl.BlockSpec((1, 3 * D), lambda r: (0, 0)),
            pl.BlockSpec((1, 3 * D), lambda r: (0, 0)),
            pl.BlockSpec((1, D), lambda r: (0, 0)),
            pl.BlockSpec((1, 1), lambda r: (0, 0)),
        ],
        out_specs=[
            pl.BlockSpec((G_RB, D), lambda r: (r, 0)),
            pl.BlockSpec((G_RB, ROW_BLK), lambda r: (r, 0)),
            pl.BlockSpec((G_RB, ROW_BLK), lambda r: (r, 0)),
        ],
        out_shape=[
            jax.ShapeDtypeStruct((NODES, D), jnp.float32),
            jax.ShapeDtypeStruct((NODES, ROW_BLK), jnp.float32),
            jax.ShapeDtypeStruct((NODES, ROW_BLK), jnp.float32),
        ],
    )(m2[0], m2[1], x, W_ihT, W_hhT, b_ih2, b_hh2, w_relT, b_rel2)


def kernel(encoder_outputs, entity_type_embeddings, linking_scores,
           utterance_mask, edge_index_0, edge_index_1, edge_index_2,
           edge_index_3, W_proj, b_proj, global_emb, W_edge, W_ih, W_hh,
           b_ih, b_hh, W_rel, b_rel):
    edge_indices = [edge_index_0, edge_index_1, edge_index_2, edge_index_3]
    src_all = jnp.concatenate(
        [edge_indices[e][0] + e * NODES for e in range(NUM_EDGE_TYPES)])
    dst_all = jnp.concatenate([edge_indices[e][1] for e in range(NUM_EDGE_TYPES)])
    npad = E_PAD - TOTAL_E
    pad_i = jnp.arange(npad, dtype=jnp.int32)
    src_p = jnp.concatenate([src_all, pad_i % CHUNK])
    dst_p = jnp.concatenate([dst_all, NODES + (pad_i % 128)])
    src2 = jnp.stack([src_p, src_p + NUM_EDGE_TYPES * NODES]).reshape(
        2, NS, CHUNKS_PER_TILE, CHUNK)
    dst2 = dst_p.reshape(NS, CHUNKS_PER_TILE, CHUNK)
    zeros_hbm = jnp.zeros((M_ROWS, HALF), jnp.float32)

    um3 = utterance_mask.reshape(B, 1, U)
    b_proj2 = b_proj.reshape(1, D)
    g2 = global_emb.reshape(1, D)
    W8 = W_edge.reshape(NUM_EDGE_TYPES, D, 2, HALF).transpose(2, 0, 1, 3).reshape(
        2 * NUM_EDGE_TYPES, D, HALF)
    W_ihT = W_ih.T
    W_hhT = W_hh.T
    b_ih2 = b_ih.reshape(1, 3 * D)
    b_hh2 = b_hh.reshape(1, 3 * D)
    w_relT = W_rel.reshape(1, D)
    b_rel2 = b_rel.reshape(1, 1)

    lp, x0 = _prep(linking_scores, um3, entity_type_embeddings,
                   encoder_outputs, W_proj, b_proj2, g2)
    x = x0.reshape(NODES, D)

    for _ in range(TIMESTEPS):
        h3 = _hproj(x, W8)
        m2 = _make_sc_agg()(h3, src2, dst2, zeros_hbm)
        x, lg, sg = _gru(m2, x, W_ihT, W_hhT, b_ih2, b_hh2, w_relT, b_rel2)

    logits = lg[:, 0].reshape(B, N + 1, 1)[:, :N]
    sig = sg[:, 0].reshape(B, N + 1, 1)[:, :N]
    return (sig, logits, lp)


# speedup vs baseline: 2.5117x; 1.0885x over previous
import functools

import jax
import jax.numpy as jnp
from jax import lax
from jax.experimental import pallas as pl
from jax.experimental.pallas import tpu as pltpu
from jax.experimental.pallas import tpu_sc as plsc

B = 64
N = 155
U = 60
D = 256
ENC = 256
NUM_EDGE_TYPES = 4
TIMESTEPS = 2
E_PER_TYPE = 80000
NODES = B * (N + 1)
TOTAL_E = NUM_EDGE_TYPES * E_PER_TYPE

HALF = D // 2
ROW_BLK = 128
NUM_ROW_BLKS = NODES // ROW_BLK
H_ROWS = 2 * NUM_EDGE_TYPES * NODES

CHUNK = 128
NS = 16
GRP = 32
CHUNKS_PER_TILE = 160
NGRP = CHUNKS_PER_TILE // GRP
E_PAD = CHUNKS_PER_TILE * CHUNK * NS
M_ROWS = NODES + 128
ZROWS_PER_TILE = M_ROWS // NS
OROWS_PER_TILE = NODES // NS
G_RB = 1248


PB = 4


def _prep_body(ls_ref, um_ref, ete_ref, enc_ref, wp_ref, bp_ref, g_ref,
               lp_ref, x0_ref):
    for b in range(PB):
        ls = ls_ref[b]
        um = um_ref[b, 0]
        vm = ls * um[None, :]
        mx = jnp.maximum(jnp.max(vm, axis=1, keepdims=True), 0.0)
        en = jnp.exp(vm - mx)
        denom = jnp.sum(en, axis=1, keepdims=True) + jnp.exp(-mx)
        s = en / denom * um[None, :]
        lp = s / (jnp.sum(s, axis=1, keepdims=True) + 1e-13)
        lp_ref[b] = lp
        r0 = jnp.max(lp, axis=1, keepdims=True)
        q = jnp.dot(lp, enc_ref[b], preferred_element_type=jnp.float32)
        init = (jnp.dot(ete_ref[b], wp_ref[0:D], preferred_element_type=jnp.float32)
                + r0 * wp_ref[D][None, :]
                + jnp.dot(q, wp_ref[D + 1:], preferred_element_type=jnp.float32)
                + bp_ref[0][None, :])
        x0_ref[b] = jnp.concatenate([init, g_ref[:]], axis=0)


def _prep(linking_scores, um3, entity_type_embeddings, encoder_outputs,
          W_proj, b_proj2, global_emb):
    return pl.pallas_call(
        _prep_body,
        grid=(B // PB,),
        in_specs=[
            pl.BlockSpec((PB, N, U), lambda b: (b, 0, 0)),
            pl.BlockSpec((PB, 1, U), lambda b: (b, 0, 0)),
            pl.BlockSpec((PB, N, D), lambda b: (b, 0, 0)),
            pl.BlockSpec((PB, U, ENC), lambda b: (b, 0, 0)),
            pl.BlockSpec((D + ENC + 1, D), lambda b: (0, 0)),
            pl.BlockSpec((1, D), lambda b: (0, 0)),
            pl.BlockSpec((1, D), lambda b: (0, 0)),
        ],
        out_specs=[
            pl.BlockSpec((PB, N, U), lambda b: (b, 0, 0)),
            pl.BlockSpec((PB, N + 1, D), lambda b: (b, 0, 0)),
        ],
        out_shape=[
            jax.ShapeDtypeStruct((B, N, U), jnp.float32),
            jax.ShapeDtypeStruct((B, N + 1, D), jnp.float32),
        ],
    )(linking_scores, um3, entity_type_embeddings, encoder_outputs,
      W_proj, b_proj2, global_emb)


HP_RB = 2496
HP_NR = NODES // HP_RB


def _hproj_body(x_ref, w_ref, h_ref):
    h_ref[...] = jnp.dot(x_ref[...], w_ref[0],
                         preferred_element_type=jnp.float32)


def _hproj(x, W8):
    return pl.pallas_call(
        _hproj_body,
        grid=(HP_NR, 2 * NUM_EDGE_TYPES),
        in_specs=[
            pl.BlockSpec((HP_RB, D), lambda r, k: (r, 0)),
            pl.BlockSpec((1, D, HALF), lambda r, k: (k, 0, 0)),
        ],
        out_specs=pl.BlockSpec(
            (HP_RB, HALF), lambda r, k: (k * HP_NR + r, 0)),
        out_shape=jax.ShapeDtypeStruct((H_ROWS, HALF), jnp.float32),
    )(x, W8)


def _sc_agg_body(h3, src_idx, dst_idx, zeros_hbm, out,
                 sidx_v, didx_v, rows0_v, rows1_v, m_sh, sem0, sem1,
                 ssem0, ssem1):
    c = lax.axis_index("c")
    s = lax.axis_index("s")
    pltpu.sync_copy(zeros_hbm.at[pl.ds(s * ZROWS_PER_TILE, ZROWS_PER_TILE)],
                    m_sh.at[pl.ds(s * ZROWS_PER_TILE, ZROWS_PER_TILE)])
    plsc.subcore_barrier()

    def outer(g, carry):
        pltpu.sync_copy(src_idx.at[c].at[s].at[pl.ds(g * GRP, GRP)], sidx_v)
        pltpu.sync_copy(dst_idx.at[s].at[pl.ds(g * GRP, GRP)], didx_v)

        pltpu.async_copy(h3.at[sidx_v.at[0]], rows0_v, sem0)

        def body(j2, c2):
            j = 2 * j2
            pltpu.async_copy(h3.at[sidx_v.at[j + 1]], rows1_v, sem1)
            pltpu.make_async_copy(h3.at[sidx_v.at[j]], rows0_v, sem0).wait()
            pltpu.sync_copy(rows0_v, m_sh.at[didx_v.at[j]], add=True)
            jn = jnp.minimum(j + 2, GRP - 1)
            pltpu.async_copy(h3.at[sidx_v.at[jn]], rows0_v, sem0)
            pltpu.make_async_copy(h3.at[sidx_v.at[j + 1]], rows1_v, sem1).wait()
            pltpu.sync_copy(rows1_v, m_sh.at[didx_v.at[j + 1]], add=True)
            return c2

        r = lax.fori_loop(0, GRP // 2, body, carry)
        pltpu.make_async_copy(h3.at[sidx_v.at[0]], rows0_v, sem0).wait()
        return r

    lax.fori_loop(0, NGRP, outer, 0)
    plsc.subcore_barrier()
    pltpu.sync_copy(m_sh.at[pl.ds(s * OROWS_PER_TILE, OROWS_PER_TILE)],
                    out.at[c].at[pl.ds(s * OROWS_PER_TILE, OROWS_PER_TILE)])


@functools.cache
def _make_sc_agg():
    @functools.partial(
        pl.kernel,
        mesh=plsc.VectorSubcoreMesh(core_axis_name="c", subcore_axis_name="s"),
        out_type=jax.ShapeDtypeStruct((2, NODES, HALF), jnp.float32),
        scratch_types=[
            pltpu.VMEM((GRP, CHUNK), jnp.int32),
            pltpu.VMEM((GRP, CHUNK), jnp.int32),
            pltpu.VMEM((CHUNK, HALF), jnp.float32),
            pltpu.VMEM((CHUNK, HALF), jnp.float32),
            pltpu.VMEM_SHARED((M_ROWS, HALF), jnp.float32),
            pltpu.SemaphoreType.DMA,
            pltpu.SemaphoreType.DMA,
            pltpu.SemaphoreType.DMA,
            pltpu.SemaphoreType.DMA,
        ],
    )
    def _sc_agg(h3, src_idx, dst_idx, zeros_hbm, out,
                sidx_v, didx_v, rows0_v, rows1_v, m_sh, sem0, sem1,
                ssem0, ssem1):
        _sc_agg_body(h3, src_idx, dst_idx, zeros_hbm, out,
                     sidx_v, didx_v, rows0_v, rows1_v, m_sh, sem0, sem1,
                     ssem0, ssem1)

    return _sc_agg


def _gru_body(mlo_ref, mhi_ref, x_ref, wih_ref, whh_ref, bih_ref, bhh_ref,
              wrel_ref, brel_ref, xn_ref, lg_ref, sg_ref):
    gi = (jnp.dot(mlo_ref[...], wih_ref[0:HALF], preferred_element_type=jnp.float32)
          + jnp.dot(mhi_ref[...], wih_ref[HALF:], preferred_element_type=jnp.float32)
          + bih_ref[0][None, :])
    gh = (jnp.dot(x_ref[...], whh_ref[...], preferred_element_type=jnp.float32)
          + bhh_ref[0][None, :])
    r = jax.nn.sigmoid(gi[:, 0:D] + gh[:, 0:D])
    z = jax.nn.sigmoid(gi[:, D:2 * D] + gh[:, D:2 * D])
    n = jnp.tanh(gi[:, 2 * D:] + r * gh[:, 2 * D:])
    xn = (1.0 - z) * n + z * x_ref[...]
    xn_ref[...] = xn
    lg = jnp.sum(xn * wrel_ref[0][None, :], axis=1, keepdims=True) + brel_ref[0, 0]
    lg_ref[...] = jnp.broadcast_to(lg, (G_RB, ROW_BLK))
    sg_ref[...] = jax.nn.sigmoid(lg_ref[...])


def _gru_h_body(mlo_ref, mhi_ref, x_ref, wih_ref, whh_ref, bih_ref, bhh_ref,
                wrel_ref, brel_ref, w8_ref, xn_ref, lg_ref, sg_ref, h_ref):
    _gru_body(mlo_ref, mhi_ref, x_ref, wih_ref, whh_ref, bih_ref, bhh_ref,
              wrel_ref, brel_ref, xn_ref, lg_ref, sg_ref)
    xn = xn_ref[...]
    for k in range(2 * NUM_EDGE_TYPES):
        h_ref[k] = jnp.dot(xn, w8_ref[k], preferred_element_type=jnp.float32)


def _gru_h(m2, x, W_ihT, W_hhT, b_ih2, b_hh2, w_relT, b_rel2, W8):
    return pl.pallas_call(
        _gru_h_body,
        grid=(NODES // G_RB,),
        in_specs=[
            pl.BlockSpec((G_RB, HALF), lambda r: (r, 0)),
            pl.BlockSpec((G_RB, HALF), lambda r: (r, 0)),
            pl.BlockSpec((G_RB, D), lambda r: (r, 0)),
            pl.BlockSpec((D, 3 * D), lambda r: (0, 0)),
            pl.BlockSpec((D, 3 * D), lambda r: (0, 0)),
            pl.BlockSpec((1, 3 * D), lambda r: (0, 0)),
            pl.BlockSpec((1, 3 * D), lambda r: (0, 0)),
            pl.BlockSpec((1, D), lambda r: (0, 0)),
            pl.BlockSpec((1, 1), lambda r: (0, 0)),
            pl.BlockSpec((2 * NUM_EDGE_TYPES, D, HALF), lambda r: (0, 0, 0)),
        ],
        out_specs=[
            pl.BlockSpec((G_RB, D), lambda r: (r, 0)),
            pl.BlockSpec((G_RB, ROW_BLK), lambda r: (r, 0)),
            pl.BlockSpec((G_RB, ROW_BLK), lambda r: (r, 0)),
            pl.BlockSpec((2 * NUM_EDGE_TYPES, G_RB, HALF), lambda r: (0, r, 0)),
        ],
        out_shape=[
            jax.ShapeDtypeStruct((NODES, D), jnp.float32),
            jax.ShapeDtypeStruct((NODES, ROW_BLK), jnp.float32),
            jax.ShapeDtypeStruct((NODES, ROW_BLK), jnp.float32),
            jax.ShapeDtypeStruct((2 * NUM_EDGE_TYPES, NODES, HALF), jnp.float32),
        ],
    )(m2[0], m2[1], x, W_ihT, W_hhT, b_ih2, b_hh2, w_relT, b_rel2, W8)


def _gru(m2, x, W_ihT, W_hhT, b_ih2, b_hh2, w_relT, b_rel2):
    return pl.pallas_call(
        _gru_body,
        grid=(NODES // G_RB,),
        in_specs=[
            pl.BlockSpec((G_RB, HALF), lambda r: (r, 0)),
            pl.BlockSpec((G_RB, HALF), lambda r: (r, 0)),
            pl.BlockSpec((G_RB, D), lambda r: (r, 0)),
            pl.BlockSpec((D, 3 * D), lambda r: (0, 0)),
            pl.BlockSpec((D, 3 * D), lambda r: (0, 0)),
            pl.BlockSpec((1, 3 * D), lambda r: (0, 0)),
            pl.BlockSpec((1, 3 * D), lambda r: (0, 0)),
            pl.BlockSpec((1, D), lambda r: (0, 0)),
            pl.BlockSpec((1, 1), lambda r: (0, 0)),
        ],
        out_specs=[
            pl.BlockSpec((G_RB, D), lambda r: (r, 0)),
            pl.BlockSpec((G_RB, ROW_BLK), lambda r: (r, 0)),
            pl.BlockSpec((G_RB, ROW_BLK), lambda r: (r, 0)),
        ],
        out_shape=[
            jax.ShapeDtypeStruct((NODES, D), jnp.float32),
            jax.ShapeDtypeStruct((NODES, ROW_BLK), jnp.float32),
            jax.ShapeDtypeStruct((NODES, ROW_BLK), jnp.float32),
        ],
    )(m2[0], m2[1], x, W_ihT, W_hhT, b_ih2, b_hh2, w_relT, b_rel2)


def kernel(encoder_outputs, entity_type_embeddings, linking_scores,
           utterance_mask, edge_index_0, edge_index_1, edge_index_2,
           edge_index_3, W_proj, b_proj, global_emb, W_edge, W_ih, W_hh,
           b_ih, b_hh, W_rel, b_rel):
    edge_indices = [edge_index_0, edge_index_1, edge_index_2, edge_index_3]
    src_all = jnp.concatenate(
        [edge_indices[e][0] + e * NODES for e in range(NUM_EDGE_TYPES)])
    dst_all = jnp.concatenate([edge_indices[e][1] for e in range(NUM_EDGE_TYPES)])
    npad = E_PAD - TOTAL_E
    pad_i = jnp.arange(npad, dtype=jnp.int32)
    src_p = jnp.concatenate([src_all, pad_i % CHUNK])
    dst_p = jnp.concatenate([dst_all, NODES + (pad_i % 128)])
    src2 = jnp.stack([src_p, src_p + NUM_EDGE_TYPES * NODES]).reshape(
        2, NS, CHUNKS_PER_TILE, CHUNK)
    dst2 = dst_p.reshape(NS, CHUNKS_PER_TILE, CHUNK)
    zeros_hbm = jnp.zeros((M_ROWS, HALF), jnp.float32)

    um3 = utterance_mask.reshape(B, 1, U)
    b_proj2 = b_proj.reshape(1, D)
    g2 = global_emb.reshape(1, D)
    W8 = W_edge.reshape(NUM_EDGE_TYPES, D, 2, HALF).transpose(2, 0, 1, 3).reshape(
        2 * NUM_EDGE_TYPES, D, HALF)
    W_ihT = W_ih.T
    W_hhT = W_hh.T
    b_ih2 = b_ih.reshape(1, 3 * D)
    b_hh2 = b_hh.reshape(1, 3 * D)
    w_relT = W_rel.reshape(1, D)
    b_rel2 = b_rel.reshape(1, 1)

    lp, x0 = _prep(linking_scores, um3, entity_type_embeddings,
                   encoder_outputs, W_proj, b_proj2, g2)
    x = x0.reshape(NODES, D)

    h3 = _hproj(x, W8)
    m2 = _make_sc_agg()(h3, src2, dst2, zeros_hbm)
    x, lg, sg, h3b = _gru_h(m2, x, W_ihT, W_hhT, b_ih2, b_hh2, w_relT,
                            b_rel2, W8)
    m2 = _make_sc_agg()(h3b.reshape(H_ROWS, HALF), src2, dst2, zeros_hbm)
    x, lg, sg = _gru(m2, x, W_ihT, W_hhT, b_ih2, b_hh2, w_relT, b_rel2)

    logits = lg[:, 0].reshape(B, N + 1, 1)[:, :N]
    sig = sg[:, 0].reshape(B, N + 1, 1)[:, :N]
    return (sig, logits, lp)


# speedup vs baseline: 2.5169x; 1.0021x over previous
import functools

import jax
import jax.numpy as jnp
from jax import lax
from jax.experimental import pallas as pl
from jax.experimental.pallas import tpu as pltpu
from jax.experimental.pallas import tpu_sc as plsc

B = 64
N = 155
U = 60
D = 256
ENC = 256
NUM_EDGE_TYPES = 4
TIMESTEPS = 2
E_PER_TYPE = 80000
NODES = B * (N + 1)
TOTAL_E = NUM_EDGE_TYPES * E_PER_TYPE

HALF = D // 2
ROW_BLK = 128
NUM_ROW_BLKS = NODES // ROW_BLK
H_ROWS = 2 * NUM_EDGE_TYPES * NODES

CHUNK = 128
NS = 16
GRP = 32
CHUNKS_PER_TILE = 160
NGRP = CHUNKS_PER_TILE // GRP
E_PAD = CHUNKS_PER_TILE * CHUNK * NS
M_ROWS = NODES + 128
ZROWS_PER_TILE = M_ROWS // NS
OROWS_PER_TILE = NODES // NS
G_RB = 1248


PB = 4


def _prep_body(ls_ref, um_ref, ete_ref, enc_ref, wp_ref, bp_ref, g_ref,
               lp_ref, x0_ref):
    for b in range(PB):
        ls = ls_ref[b]
        um = um_ref[b, 0]
        vm = ls * um[None, :]
        mx = jnp.maximum(jnp.max(vm, axis=1, keepdims=True), 0.0)
        en = jnp.exp(vm - mx)
        denom = jnp.sum(en, axis=1, keepdims=True) + jnp.exp(-mx)
        s = en / denom * um[None, :]
        lp = s / (jnp.sum(s, axis=1, keepdims=True) + 1e-13)
        lp_ref[b] = lp
        r0 = jnp.max(lp, axis=1, keepdims=True)
        q = jnp.dot(lp, enc_ref[b], preferred_element_type=jnp.float32)
        init = (jnp.dot(ete_ref[b], wp_ref[0:D], preferred_element_type=jnp.float32)
                + r0 * wp_ref[D][None, :]
                + jnp.dot(q, wp_ref[D + 1:], preferred_element_type=jnp.float32)
                + bp_ref[0][None, :])
        x0_ref[b] = jnp.concatenate([init, g_ref[:]], axis=0)


def _prep(linking_scores, um3, entity_type_embeddings, encoder_outputs,
          W_proj, b_proj2, global_emb):
    return pl.pallas_call(
        _prep_body,
        grid=(B // PB,),
        in_specs=[
            pl.BlockSpec((PB, N, U), lambda b: (b, 0, 0)),
            pl.BlockSpec((PB, 1, U), lambda b: (b, 0, 0)),
            pl.BlockSpec((PB, N, D), lambda b: (b, 0, 0)),
            pl.BlockSpec((PB, U, ENC), lambda b: (b, 0, 0)),
            pl.BlockSpec((D + ENC + 1, D), lambda b: (0, 0)),
            pl.BlockSpec((1, D), lambda b: (0, 0)),
            pl.BlockSpec((1, D), lambda b: (0, 0)),
        ],
        out_specs=[
            pl.BlockSpec((PB, N, U), lambda b: (b, 0, 0)),
            pl.BlockSpec((PB, N + 1, D), lambda b: (b, 0, 0)),
        ],
        out_shape=[
            jax.ShapeDtypeStruct((B, N, U), jnp.float32),
            jax.ShapeDtypeStruct((B, N + 1, D), jnp.float32),
        ],
    )(linking_scores, um3, entity_type_embeddings, encoder_outputs,
      W_proj, b_proj2, global_emb)


HP_RB = 2496
HP_NR = NODES // HP_RB


def _hproj_body(x_ref, w_ref, h_ref):
    h_ref[...] = jnp.dot(x_ref[...], w_ref[0],
                         preferred_element_type=jnp.float32)


def _hproj(x, W8):
    return pl.pallas_call(
        _hproj_body,
        grid=(HP_NR, 2 * NUM_EDGE_TYPES),
        in_specs=[
            pl.BlockSpec((HP_RB, D), lambda r, k: (r, 0)),
            pl.BlockSpec((1, D, HALF), lambda r, k: (k, 0, 0)),
        ],
        out_specs=pl.BlockSpec(
            (HP_RB, HALF), lambda r, k: (k * HP_NR + r, 0)),
        out_shape=jax.ShapeDtypeStruct((H_ROWS, HALF), jnp.float32),
    )(x, W8)


def _sc_agg_body(h3, src_idx, dst_idx, zeros_hbm, out,
                 sidx_v, didx_v, rows0_v, rows1_v, m_sh, sem0, sem1):
    c = lax.axis_index("c")
    s = lax.axis_index("s")
    pltpu.sync_copy(zeros_hbm.at[pl.ds(s * ZROWS_PER_TILE, ZROWS_PER_TILE)],
                    m_sh.at[pl.ds(s * ZROWS_PER_TILE, ZROWS_PER_TILE)])
    plsc.subcore_barrier()

    def outer(g, carry):
        pltpu.sync_copy(src_idx.at[c].at[s].at[pl.ds(g * GRP, GRP)], sidx_v)
        pltpu.sync_copy(dst_idx.at[s].at[pl.ds(g * GRP, GRP)], didx_v)

        pltpu.async_copy(h3.at[sidx_v.at[0]], rows0_v, sem0)

        def body(j2, c2):
            j = 2 * j2
            pltpu.async_copy(h3.at[sidx_v.at[j + 1]], rows1_v, sem1)
            pltpu.make_async_copy(h3.at[sidx_v.at[j]], rows0_v, sem0).wait()
            pltpu.sync_copy(rows0_v, m_sh.at[didx_v.at[j]], add=True)
            jn = jnp.minimum(j + 2, GRP - 1)
            pltpu.async_copy(h3.at[sidx_v.at[jn]], rows0_v, sem0)
            pltpu.make_async_copy(h3.at[sidx_v.at[j + 1]], rows1_v, sem1).wait()
            pltpu.sync_copy(rows1_v, m_sh.at[didx_v.at[j + 1]], add=True)
            return c2

        r = lax.fori_loop(0, GRP // 2, body, carry)
        pltpu.make_async_copy(h3.at[sidx_v.at[0]], rows0_v, sem0).wait()
        return r

    lax.fori_loop(0, NGRP, outer, 0)
    plsc.subcore_barrier()
    pltpu.sync_copy(m_sh.at[pl.ds(s * OROWS_PER_TILE, OROWS_PER_TILE)],
                    out.at[c].at[pl.ds(s * OROWS_PER_TILE, OROWS_PER_TILE)])


@functools.cache
def _make_sc_agg():
    @functools.partial(
        pl.kernel,
        mesh=plsc.VectorSubcoreMesh(core_axis_name="c", subcore_axis_name="s"),
        out_type=jax.ShapeDtypeStruct((2, NODES, HALF), jnp.float32),
        scratch_types=[
            pltpu.VMEM((GRP, CHUNK), jnp.int32),
            pltpu.VMEM((GRP, CHUNK), jnp.int32),
            pltpu.VMEM((CHUNK, HALF), jnp.float32),
            pltpu.VMEM((CHUNK, HALF), jnp.float32),
            pltpu.VMEM_SHARED((M_ROWS, HALF), jnp.float32),
            pltpu.SemaphoreType.DMA,
            pltpu.SemaphoreType.DMA,
        ],
    )
    def _sc_agg(h3, src_idx, dst_idx, zeros_hbm, out,
                sidx_v, didx_v, rows0_v, rows1_v, m_sh, sem0, sem1):
        _sc_agg_body(h3, src_idx, dst_idx, zeros_hbm, out,
                     sidx_v, didx_v, rows0_v, rows1_v, m_sh, sem0, sem1)

    return _sc_agg


def _gru_body(mlo_ref, mhi_ref, x_ref, wih_ref, whh_ref, bih_ref, bhh_ref,
              wrel_ref, brel_ref, xn_ref, lg_ref, sg_ref):
    gi = (jnp.dot(mlo_ref[...], wih_ref[0:HALF], preferred_element_type=jnp.float32)
          + jnp.dot(mhi_ref[...], wih_ref[HALF:], preferred_element_type=jnp.float32)
          + bih_ref[0][None, :])
    gh = (jnp.dot(x_ref[...], whh_ref[...], preferred_element_type=jnp.float32)
          + bhh_ref[0][None, :])
    r = jax.nn.sigmoid(gi[:, 0:D] + gh[:, 0:D])
    z = jax.nn.sigmoid(gi[:, D:2 * D] + gh[:, D:2 * D])
    n = jnp.tanh(gi[:, 2 * D:] + r * gh[:, 2 * D:])
    xn = (1.0 - z) * n + z * x_ref[...]
    xn_ref[...] = xn
    lg = jnp.sum(xn * wrel_ref[0][None, :], axis=1, keepdims=True) + brel_ref[0, 0]
    lg_ref[...] = jnp.broadcast_to(lg, (G_RB, ROW_BLK))
    sg_ref[...] = jax.nn.sigmoid(lg_ref[...])


def _gru_h_body(mlo_ref, mhi_ref, x_ref, wih_ref, whh_ref, bih_ref, bhh_ref,
                wrel_ref, brel_ref, w8_ref, xn_ref, lg_ref, sg_ref, h_ref):
    _gru_body(mlo_ref, mhi_ref, x_ref, wih_ref, whh_ref, bih_ref, bhh_ref,
              wrel_ref, brel_ref, xn_ref, lg_ref, sg_ref)
    xn = xn_ref[...]
    for k in range(2 * NUM_EDGE_TYPES):
        h_ref[k] = jnp.dot(xn, w8_ref[k], preferred_element_type=jnp.float32)


def _gru_h(m2, x, W_ihT, W_hhT, b_ih2, b_hh2, w_relT, b_rel2, W8):
    return pl.pallas_call(
        _gru_h_body,
        grid=(NODES // G_RB,),
        in_specs=[
            pl.BlockSpec((G_RB, HALF), lambda r: (r, 0)),
            pl.BlockSpec((G_RB, HALF), lambda r: (r, 0)),
            pl.BlockSpec((G_RB, D), lambda r: (r, 0)),
            pl.BlockSpec((D, 3 * D), lambda r: (0, 0)),
            pl.BlockSpec((D, 3 * D), lambda r: (0, 0)),
            pl.BlockSpec((1, 3 * D), lambda r: (0, 0)),
            pl.BlockSpec((1, 3 * D), lambda r: (0, 0)),
            pl.BlockSpec((1, D), lambda r: (0, 0)),
            pl.BlockSpec((1, 1), lambda r: (0, 0)),
            pl.BlockSpec((2 * NUM_EDGE_TYPES, D, HALF), lambda r: (0, 0, 0)),
        ],
        out_specs=[
            pl.BlockSpec((G_RB, D), lambda r: (r, 0)),
            pl.BlockSpec((G_RB, ROW_BLK), lambda r: (r, 0)),
            pl.BlockSpec((G_RB, ROW_BLK), lambda r: (r, 0)),
            pl.BlockSpec((2 * NUM_EDGE_TYPES, G_RB, HALF), lambda r: (0, r, 0)),
        ],
        out_shape=[
            jax.ShapeDtypeStruct((NODES, D), jnp.float32),
            jax.ShapeDtypeStruct((NODES, ROW_BLK), jnp.float32),
            jax.ShapeDtypeStruct((NODES, ROW_BLK), jnp.float32),
            jax.ShapeDtypeStruct((2 * NUM_EDGE_TYPES, NODES, HALF), jnp.float32),
        ],
    )(m2[0], m2[1], x, W_ihT, W_hhT, b_ih2, b_hh2, w_relT, b_rel2, W8)


def _gru(m2, x, W_ihT, W_hhT, b_ih2, b_hh2, w_relT, b_rel2):
    return pl.pallas_call(
        _gru_body,
        grid=(NODES // G_RB,),
        in_specs=[
            pl.BlockSpec((G_RB, HALF), lambda r: (r, 0)),
            pl.BlockSpec((G_RB, HALF), lambda r: (r, 0)),
            pl.BlockSpec((G_RB, D), lambda r: (r, 0)),
            pl.BlockSpec((D, 3 * D), lambda r: (0, 0)),
            pl.BlockSpec((D, 3 * D), lambda r: (0, 0)),
            pl.BlockSpec((1, 3 * D), lambda r: (0, 0)),
            pl.BlockSpec((1, 3 * D), lambda r: (0, 0)),
            pl.BlockSpec((1, D), lambda r: (0, 0)),
            pl.BlockSpec((1, 1), lambda r: (0, 0)),
        ],
        out_specs=[
            pl.BlockSpec((G_RB, D), lambda r: (r, 0)),
            pl.BlockSpec((G_RB, ROW_BLK), lambda r: (r, 0)),
            pl.BlockSpec((G_RB, ROW_BLK), lambda r: (r, 0)),
        ],
        out_shape=[
            jax.ShapeDtypeStruct((NODES, D), jnp.float32),
            jax.ShapeDtypeStruct((NODES, ROW_BLK), jnp.float32),
            jax.ShapeDtypeStruct((NODES, ROW_BLK), jnp.float32),
        ],
    )(m2[0], m2[1], x, W_ihT, W_hhT, b_ih2, b_hh2, w_relT, b_rel2)


def kernel(encoder_outputs, entity_type_embeddings, linking_scores,
           utterance_mask, edge_index_0, edge_index_1, edge_index_2,
           edge_index_3, W_proj, b_proj, global_emb, W_edge, W_ih, W_hh,
           b_ih, b_hh, W_rel, b_rel):
    edge_indices = [edge_index_0, edge_index_1, edge_index_2, edge_index_3]
    src_all = jnp.concatenate(
        [edge_indices[e][0] + e * NODES for e in range(NUM_EDGE_TYPES)])
    dst_all = jnp.concatenate([edge_indices[e][1] for e in range(NUM_EDGE_TYPES)])
    npad = E_PAD - TOTAL_E
    pad_i = jnp.arange(npad, dtype=jnp.int32)
    src_p = jnp.concatenate([src_all, pad_i % CHUNK])
    dst_p = jnp.concatenate([dst_all, NODES + (pad_i % 128)])
    src2 = jnp.stack([src_p, src_p + NUM_EDGE_TYPES * NODES]).reshape(
        2, NS, CHUNKS_PER_TILE, CHUNK)
    dst2 = dst_p.reshape(NS, CHUNKS_PER_TILE, CHUNK)
    zeros_hbm = jnp.zeros((M_ROWS, HALF), jnp.float32)

    um3 = utterance_mask.reshape(B, 1, U)
    b_proj2 = b_proj.reshape(1, D)
    g2 = global_emb.reshape(1, D)
    W8 = W_edge.reshape(NUM_EDGE_TYPES, D, 2, HALF).transpose(2, 0, 1, 3).reshape(
        2 * NUM_EDGE_TYPES, D, HALF)
    W_ihT = W_ih.T
    W_hhT = W_hh.T
    b_ih2 = b_ih.reshape(1, 3 * D)
    b_hh2 = b_hh.reshape(1, 3 * D)
    w_relT = W_rel.reshape(1, D)
    b_rel2 = b_rel.reshape(1, 1)

    lp, x0 = _prep(linking_scores, um3, entity_type_embeddings,
                   encoder_outputs, W_proj, b_proj2, g2)
    x = x0.reshape(NODES, D)

    h3 = _hproj(x, W8)
    m2 = _make_sc_agg()(h3, src2, dst2, zeros_hbm)
    x, lg, sg, h3b = _gru_h(m2, x, W_ihT, W_hhT, b_ih2, b_hh2, w_relT,
                            b_rel2, W8)
    m2 = _make_sc_agg()(h3b.reshape(H_ROWS, HALF), src2, dst2, zeros_hbm)
    x, lg, sg = _gru(m2, x, W_ihT, W_hhT, b_ih2, b_hh2, w_relT, b_rel2)

    logits = lg[:, 0].reshape(B, N + 1, 1)[:, :N]
    sig = sg[:, 0].reshape(B, N + 1, 1)[:, :N]
    return (sig, logits, lp)
